# Initial kernel scaffold; baseline (speedup 1.0000x reference)
import jax
import jax.numpy as jnp
from jax.experimental import pallas as pl


def kernel(x1, edge_index1, batch1, drug1, x2, edge_index2, batch2, drug2, ddi_type, W_node, b_node, W_g1, b_g1, W_g2, b_g2, W_pat, W_m0, b_m0, W_m1, b_m1, W_m2, b_m2, W_m3, b_m3, emb):
    raise NotImplementedError("write your pallas kernel here")


# speedup vs baseline: 10.2508x; 10.2508x over previous
import functools

import jax
import jax.numpy as jnp
from jax import lax
from jax.experimental import pallas as pl
from jax.experimental.pallas import tpu as pltpu
import jax.experimental.pallas.tpu_sc as plsc

N = 10000
NP = 10240
E = 320000
D = 128
B = 256
P = 8
NDDI = 86
NDRUGS = 1000

NC = 2
NS = 16
EPT = E // NS
K = 80
NIT = EPT // K
RPT = NP // NS

_MESH = plsc.VectorSubcoreMesh(
    core_axis_name="c", subcore_axis_name="s", num_cores=NC, num_subcores=NS)


@functools.partial(
    pl.kernel,
    out_type=jax.ShapeDtypeStruct((NC * NP, D), jnp.float32),
    mesh=_MESH,
    scratch_types=[
        pltpu.VMEM((K,), jnp.int32),
        pltpu.VMEM((K, D), jnp.float32),
        pltpu.VMEM_SHARED((NP, D), jnp.float32),
    ],
)
def _sc_degree(dst_hbm, ones_hbm, zz_hbm, out_hbm, dst_v, ones_v, deg_sh):
    c = lax.axis_index("c")
    s = lax.axis_index("s")
    pltpu.sync_copy(ones_hbm, ones_v)
    pltpu.sync_copy(zz_hbm, deg_sh.at[pl.ds(s * RPT, RPT)])
    plsc.subcore_barrier()

    def body(i, carry):
        base = c * E + s * EPT + i * K
        pltpu.sync_copy(dst_hbm.at[pl.ds(base, K)], dst_v)
        pltpu.sync_copy(ones_v, deg_sh.at[dst_v], add=True)
        return carry

    lax.fori_loop(0, NIT, body, 0)
    plsc.subcore_barrier()
    pltpu.sync_copy(deg_sh.at[pl.ds(s * RPT, RPT)],
                    out_hbm.at[pl.ds(c * NP + s * RPT, RPT)])


@functools.partial(
    pl.kernel,
    out_type=jax.ShapeDtypeStruct((NC * NP, D), jnp.float32),
    mesh=_MESH,
    scratch_types=[
        pltpu.VMEM((K,), jnp.int32),
        pltpu.VMEM((K,), jnp.int32),
        pltpu.VMEM((K, D), jnp.float32),
        pltpu.VMEM_SHARED((NP, D), jnp.float32),
        pltpu.SemaphoreType.DMA,
    ],
)
def _sc_conv(u_hbm, src_hbm, dst_hbm, zz_hbm, out_hbm,
             src_v, dst_v, rows_v, acc_sh, sem):
    c = lax.axis_index("c")
    s = lax.axis_index("s")
    pltpu.sync_copy(zz_hbm, acc_sh.at[pl.ds(s * RPT, RPT)])
    plsc.subcore_barrier()

    def body(i, carry):
        base = c * E + s * EPT + i * K
        pltpu.sync_copy(src_hbm.at[pl.ds(base, K)], src_v)
        pltpu.sync_copy(dst_hbm.at[pl.ds(base, K)], dst_v)
        pltpu.async_copy(u_hbm.at[src_v], rows_v, sem).wait()
        pltpu.sync_copy(rows_v, acc_sh.at[dst_v], add=True)
        return carry

    lax.fori_loop(0, NIT, body, 0)
    plsc.subcore_barrier()
    pltpu.sync_copy(acc_sh.at[pl.ds(s * RPT, RPT)],
                    out_hbm.at[pl.ds(c * NP + s * RPT, RPT)])


_RB = 2048
_NB = 2000


def _tc_pre_body(x_ref, deg_ref, wn_ref, bn_ref, wg1_ref, u_ref, dinv_ref):
    h = jnp.dot(x_ref[...], wn_ref[...], preferred_element_type=jnp.float32)
    h = h + bn_ref[...]
    z = jnp.dot(h, wg1_ref[...], preferred_element_type=jnp.float32)
    dinv = lax.rsqrt(deg_ref[...][:, 0:8] + 1.0)
    dinv_ref[...] = dinv
    u_ref[...] = z * dinv[:, 0:1]


def _tc_mid_body(acc_ref, u_ref, dinv_ref, bg_ref, wg2_ref, u2_ref):
    dv = dinv_ref[...][:, 0:1]
    a = dv * (acc_ref[...] + u_ref[...]) + bg_ref[...]
    z2 = jnp.dot(a, wg2_ref[...], preferred_element_type=jnp.float32)
    u2_ref[...] = z2 * dv


def _tc_pool_body(acc_ref, u_ref, dinv_ref, bg_ref, batch_ref, wpat_ref,
                  out_ref, pool_ref):
    i = pl.program_id(0)
    hf = dinv_ref[...][:, 0:1] * (acc_ref[...] + u_ref[...]) + bg_ref[...]
    seg = (lax.broadcasted_iota(jnp.int32, (B, _NB), 0)
           == batch_ref[0]).astype(jnp.float32)
    part_out = jnp.dot(seg, hf, preferred_element_type=jnp.float32,
                       precision=lax.Precision.HIGHEST)
    sc = jnp.dot(hf, wpat_ref[...], preferred_element_type=jnp.float32)
    sc = sc - jnp.max(sc, axis=1, keepdims=True)
    ex = jnp.exp(sc)
    attn = ex / jnp.sum(ex, axis=1, keepdims=True)
    pools = [
        jnp.dot(seg, attn[:, p:p + 1] * hf, preferred_element_type=jnp.float32,
                precision=lax.Precision.HIGHEST)
        for p in range(P)
    ]
    pstk = jnp.stack(pools, axis=1)

    @pl.when(i == 0)
    def _():
        out_ref[...] = part_out
        pool_ref[...] = pstk

    @pl.when(i > 0)
    def _():
        out_ref[...] = out_ref[...] + part_out
        pool_ref[...] = pool_ref[...] + pstk

    @pl.when(i == (N // _NB) - 1)
    def _():
        pv = pool_ref[...]
        nrm = jnp.sqrt(jnp.sum(pv * pv, axis=-1, keepdims=True))
        pool_ref[...] = pv / jnp.maximum(nrm, 1e-12)


def _tc_head_body(o1_ref, o2_ref, p1_ref, p2_ref, ddi_ref, d1_ref, d2_ref,
                  wm0a_ref, wm0b_ref, wm0c_ref, bm0_ref, wm1_ref, bm1_ref,
                  wm2_ref, bm2_ref, wm3_ref, bm3_ref, emb_ref,
                  score_ref, eout_ref, es1_ref, es2_ref):
    oh = (lax.broadcasted_iota(jnp.int32, (B, NDDI), 1)
          == ddi_ref[...]).astype(jnp.float32)
    h = (jnp.dot(o1_ref[...], wm0a_ref[...], preferred_element_type=jnp.float32)
         + jnp.dot(o2_ref[...], wm0b_ref[...], preferred_element_type=jnp.float32)
         + jnp.dot(oh, wm0c_ref[...], preferred_element_type=jnp.float32)
         + bm0_ref[...])
    h = jnp.maximum(
        jnp.dot(h, wm1_ref[...], preferred_element_type=jnp.float32)
        + bm1_ref[...], 0.0)
    h = jnp.maximum(
        jnp.dot(h, wm2_ref[...], preferred_element_type=jnp.float32)
        + bm2_ref[...], 0.0)
    score_ref[...] = (
        jnp.dot(h, wm3_ref[...], preferred_element_type=jnp.float32)
        + bm3_ref[...])

    es1_ref[...] = jnp.concatenate(
        [o1_ref[...][:, None, :], p1_ref[...]], axis=1)
    es2_ref[...] = jnp.concatenate(
        [o2_ref[...][:, None, :], p2_ref[...]], axis=1)
    eout_ref[...] = emb_ref[...]

    def wr1(b, carry):
        d = d1_ref[0, b]
        eout_ref[pl.ds(d, 1)] = es1_ref[pl.ds(b, 1)]
        return carry

    lax.fori_loop(0, B, wr1, 0)

    def wr2(b, carry):
        d = d2_ref[0, b]
        eout_ref[pl.ds(d, 1)] = es2_ref[pl.ds(b, 1)]
        return carry

    lax.fori_loop(0, B, wr2, 0)


def _row_blocked(n_arrays, nrows):
    grid = (nrows // _RB,)
    row_spec = lambda w: pl.BlockSpec((_RB, w), lambda i: (i, 0))
    full = lambda a, b: pl.BlockSpec((a, b), lambda i: (0, 0))
    return grid, row_spec, full


def _tc_pre(x_all, deg_all, wn, bn, wg1):
    grid, row, full = _row_blocked(2, NC * NP)
    return pl.pallas_call(
        _tc_pre_body,
        grid=grid,
        in_specs=[row(D), row(D), full(D, D), full(1, D), full(D, D)],
        out_specs=[row(D), row(8)],
        out_shape=[
            jax.ShapeDtypeStruct((NC * NP, D), jnp.float32),
            jax.ShapeDtypeStruct((NC * NP, 8), jnp.float32),
        ],
    )(x_all, deg_all, wn, bn, wg1)


def _tc_mid(acc_all, u_all, dinv_all, bg1, wg2):
    grid, row, full = _row_blocked(2, NC * NP)
    return pl.pallas_call(
        _tc_mid_body,
        grid=grid,
        in_specs=[row(D), row(D), row(8), full(1, D), full(D, D)],
        out_specs=row(D),
        out_shape=jax.ShapeDtypeStruct((NC * NP, D), jnp.float32),
    )(acc_all, u_all, dinv_all, bg1, wg2)


def _tc_pool(acc2, u2, dinv, bg2, batch2d, wpat):
    fix = lambda a, b: pl.BlockSpec((a, b), lambda i: (0, 0))
    return pl.pallas_call(
        _tc_pool_body,
        grid=(N // _NB,),
        in_specs=[
            pl.BlockSpec((_NB, D), lambda i: (i, 0)),
            pl.BlockSpec((_NB, D), lambda i: (i, 0)),
            pl.BlockSpec((_NB, 8), lambda i: (i, 0)),
            fix(1, D),
            pl.BlockSpec((1, 1, _NB), lambda i: (i, 0, 0)),
            fix(D, P),
        ],
        out_specs=[
            fix(B, D),
            pl.BlockSpec((B, P, D), lambda i: (0, 0, 0)),
        ],
        out_shape=[
            jax.ShapeDtypeStruct((B, D), jnp.float32),
            jax.ShapeDtypeStruct((B, P, D), jnp.float32),
        ],
    )(acc2, u2, dinv, bg2, batch2d, wpat)


def _tc_head(o1, o2, p1, p2, ddi2d, d1, d2, wm0a, wm0b, wm0c, bm0,
             wm1, bm1, wm2, bm2, wm3, bm3, emb):
    smem_spec = pl.BlockSpec(memory_space=pltpu.SMEM)
    any_spec = pl.BlockSpec()
    return pl.pallas_call(
        _tc_head_body,
        in_specs=[any_spec] * 5 + [smem_spec, smem_spec] + [any_spec] * 11,
        out_specs=[any_spec, any_spec],
        out_shape=[
            jax.ShapeDtypeStruct((B, 1), jnp.float32),
            jax.ShapeDtypeStruct((NDRUGS, P + 1, D), jnp.float32),
        ],
        scratch_shapes=[
            pltpu.VMEM((B, P + 1, D), jnp.float32),
            pltpu.VMEM((B, P + 1, D), jnp.float32),
        ],
    )(o1, o2, p1, p2, ddi2d, d1, d2, wm0a, wm0b, wm0c, bm0,
      wm1, bm1, wm2, bm2, wm3, bm3, emb)


def kernel(x1, edge_index1, batch1, drug1, x2, edge_index2, batch2, drug2,
           ddi_type, W_node, b_node, W_g1, b_g1, W_g2, b_g2, W_pat,
           W_m0, b_m0, W_m1, b_m1, W_m2, b_m2, W_m3, b_m3, emb):
    i32 = jnp.int32
    src_all = jnp.concatenate(
        [edge_index1[0].astype(i32), edge_index2[0].astype(i32) + NP])
    dst_all = jnp.concatenate(
        [edge_index1[1].astype(i32), edge_index2[1].astype(i32)])
    xpad = jnp.zeros((NP - N, D), jnp.float32)
    x_all = jnp.concatenate([x1, xpad, x2, xpad], axis=0)

    onesD = jnp.ones((K, D), jnp.float32)
    zzD = jnp.zeros((RPT, D), jnp.float32)

    deg_all = _sc_degree(dst_all, onesD, zzD)

    bn = b_node.reshape(1, D)
    bg1 = b_g1.reshape(1, D)
    bg2 = b_g2.reshape(1, D)
    u_all, dinv_all = _tc_pre(x_all, deg_all, W_node, bn, W_g1)

    acc1 = _sc_conv(u_all, src_all, dst_all, zzD)
    u2_all = _tc_mid(acc1, u_all, dinv_all, bg1, W_g2)
    acc2 = _sc_conv(u2_all, src_all, dst_all, zzD)

    b1_2d = batch1.astype(i32).reshape(N // _NB, 1, _NB)
    b2_2d = batch2.astype(i32).reshape(N // _NB, 1, _NB)
    out1, pool1 = _tc_pool(acc2[:N], u2_all[:N], dinv_all[:N], bg2,
                           b1_2d, W_pat)
    out2, pool2 = _tc_pool(acc2[NP:NP + N], u2_all[NP:NP + N],
                           dinv_all[NP:NP + N], bg2, b2_2d, W_pat)

    score2d, emb_out = _tc_head(
        out1, out2, pool1, pool2,
        ddi_type.astype(i32).reshape(B, 1),
        drug1.astype(i32).reshape(1, B),
        drug2.astype(i32).reshape(1, B),
        W_m0[:D], W_m0[D:2 * D], W_m0[2 * D:], b_m0.reshape(1, D),
        W_m1, b_m1.reshape(1, D), W_m2, b_m2.reshape(1, D),
        W_m3, b_m3.reshape(1, 1), emb)

    return score2d[:, 0], emb_out


# speedup vs baseline: 14.0282x; 1.3685x over previous
import functools

import jax
import jax.numpy as jnp
from jax import lax
from jax.experimental import pallas as pl
from jax.experimental.pallas import tpu as pltpu
import jax.experimental.pallas.tpu_sc as plsc

N = 10000
NP = 10240
E = 320000
D = 128
B = 256
P = 8
NDDI = 86
NDRUGS = 1000

NC = 2
NS = 16
EPT = E // NS
K = 80
NIT = EPT // K
RPT = NP // NS

_MESH = plsc.VectorSubcoreMesh(
    core_axis_name="c", subcore_axis_name="s", num_cores=NC, num_subcores=NS)


@functools.partial(
    pl.kernel,
    out_type=jax.ShapeDtypeStruct((NC * NP, D), jnp.float32),
    mesh=_MESH,
    scratch_types=[
        pltpu.VMEM((K,), jnp.int32),
        pltpu.VMEM((K, D), jnp.float32),
        pltpu.VMEM_SHARED((NP, D), jnp.float32),
    ],
)
def _sc_degree(dst_hbm, ones_hbm, zz_hbm, out_hbm, dst_v, ones_v, deg_sh):
    c = lax.axis_index("c")
    s = lax.axis_index("s")
    pltpu.sync_copy(ones_hbm, ones_v)
    pltpu.sync_copy(zz_hbm, deg_sh.at[pl.ds(s * RPT, RPT)])
    plsc.subcore_barrier()

    def body(i, carry):
        base = c * E + s * EPT + i * K
        pltpu.sync_copy(dst_hbm.at[pl.ds(base, K)], dst_v)
        pltpu.sync_copy(ones_v, deg_sh.at[dst_v], add=True)
        return carry

    lax.fori_loop(0, NIT, body, 0)
    plsc.subcore_barrier()
    pltpu.sync_copy(deg_sh.at[pl.ds(s * RPT, RPT)],
                    out_hbm.at[pl.ds(c * NP + s * RPT, RPT)])


@functools.partial(
    pl.kernel,
    out_type=jax.ShapeDtypeStruct((NC * NP, D), jnp.float32),
    mesh=_MESH,
    scratch_types=[
        pltpu.VMEM((K,), jnp.int32),
        pltpu.VMEM((K,), jnp.int32),
        pltpu.VMEM((K,), jnp.int32),
        pltpu.VMEM((K,), jnp.int32),
        pltpu.VMEM((K, D), jnp.float32),
        pltpu.VMEM((K, D), jnp.float32),
        pltpu.VMEM_SHARED((NP, D), jnp.float32),
        pltpu.SemaphoreType.DMA,
        pltpu.SemaphoreType.DMA,
    ],
)
def _sc_conv(u_hbm, src_hbm, dst_hbm, zz_hbm, out_hbm,
             srcA, dstA, srcB, dstB, rowsA, rowsB, acc_sh, semA, semB):
    c = lax.axis_index("c")
    s = lax.axis_index("s")
    pltpu.sync_copy(zz_hbm, acc_sh.at[pl.ds(s * RPT, RPT)])
    plsc.subcore_barrier()

    base0 = c * E + s * EPT
    pltpu.sync_copy(src_hbm.at[pl.ds(base0, K)], srcA)
    pltpu.sync_copy(dst_hbm.at[pl.ds(base0, K)], dstA)

    def body(j, carry):
        cpA = pltpu.async_copy(u_hbm.at[srcA], rowsA, semA)
        b1 = base0 + (2 * j + 1) * K
        pltpu.sync_copy(src_hbm.at[pl.ds(b1, K)], srcB)
        pltpu.sync_copy(dst_hbm.at[pl.ds(b1, K)], dstB)
        cpB = pltpu.async_copy(u_hbm.at[srcB], rowsB, semB)
        cpA.wait()
        pltpu.sync_copy(rowsA, acc_sh.at[dstA], add=True)
        b2 = base0 + lax.rem(2 * j + 2, NIT) * K
        pltpu.sync_copy(src_hbm.at[pl.ds(b2, K)], srcA)
        pltpu.sync_copy(dst_hbm.at[pl.ds(b2, K)], dstA)
        cpB.wait()
        pltpu.sync_copy(rowsB, acc_sh.at[dstB], add=True)
        return carry

    lax.fori_loop(0, NIT // 2, body, 0)

    plsc.subcore_barrier()
    pltpu.sync_copy(acc_sh.at[pl.ds(s * RPT, RPT)],
                    out_hbm.at[pl.ds(c * NP + s * RPT, RPT)])


_RB = 2048
_NB = 2000


def _tc_pre_body(x_ref, deg_ref, wn_ref, bn_ref, wg1_ref, u_ref, dinv_ref):
    h = jnp.dot(x_ref[...], wn_ref[...], preferred_element_type=jnp.float32)
    h = h + bn_ref[...]
    z = jnp.dot(h, wg1_ref[...], preferred_element_type=jnp.float32)
    dinv = lax.rsqrt(deg_ref[...][:, 0:8] + 1.0)
    dinv_ref[...] = dinv
    u_ref[...] = z * dinv[:, 0:1]


def _tc_mid_body(acc_ref, u_ref, dinv_ref, bg_ref, wg2_ref, u2_ref):
    dv = dinv_ref[...][:, 0:1]
    a = dv * (acc_ref[...] + u_ref[...]) + bg_ref[...]
    z2 = jnp.dot(a, wg2_ref[...], preferred_element_type=jnp.float32)
    u2_ref[...] = z2 * dv


def _tc_pool_body(acc_ref, u_ref, dinv_ref, bg_ref, batch_ref, wpat_ref,
                  out_ref, pool_ref):
    i = pl.program_id(0)
    hf = dinv_ref[...][:, 0:1] * (acc_ref[...] + u_ref[...]) + bg_ref[...]
    seg = (lax.broadcasted_iota(jnp.int32, (B, _NB), 0)
           == batch_ref[0]).astype(jnp.float32)
    part_out = jnp.dot(seg, hf, preferred_element_type=jnp.float32,
                       precision=lax.Precision.HIGHEST)
    sc = jnp.dot(hf, wpat_ref[...], preferred_element_type=jnp.float32)
    sc = sc - jnp.max(sc, axis=1, keepdims=True)
    ex = jnp.exp(sc)
    attn = ex / jnp.sum(ex, axis=1, keepdims=True)
    pools = [
        jnp.dot(seg, attn[:, p:p + 1] * hf, preferred_element_type=jnp.float32,
                precision=lax.Precision.HIGHEST)
        for p in range(P)
    ]
    pstk = jnp.stack(pools, axis=1)

    @pl.when(i == 0)
    def _():
        out_ref[...] = part_out
        pool_ref[...] = pstk

    @pl.when(i > 0)
    def _():
        out_ref[...] = out_ref[...] + part_out
        pool_ref[...] = pool_ref[...] + pstk

    @pl.when(i == (N // _NB) - 1)
    def _():
        pv = pool_ref[...]
        nrm = jnp.sqrt(jnp.sum(pv * pv, axis=-1, keepdims=True))
        pool_ref[...] = pv / jnp.maximum(nrm, 1e-12)


def _tc_head_body(o1_ref, o2_ref, p1_ref, p2_ref, ddi_ref, d1_ref, d2_ref,
                  wm0a_ref, wm0b_ref, wm0c_ref, bm0_ref, wm1_ref, bm1_ref,
                  wm2_ref, bm2_ref, wm3_ref, bm3_ref, emb_ref,
                  score_ref, eout_ref, es1_ref, es2_ref):
    oh = (lax.broadcasted_iota(jnp.int32, (B, NDDI), 1)
          == ddi_ref[...]).astype(jnp.float32)
    h = (jnp.dot(o1_ref[...], wm0a_ref[...], preferred_element_type=jnp.float32)
         + jnp.dot(o2_ref[...], wm0b_ref[...], preferred_element_type=jnp.float32)
         + jnp.dot(oh, wm0c_ref[...], preferred_element_type=jnp.float32)
         + bm0_ref[...])
    h = jnp.maximum(
        jnp.dot(h, wm1_ref[...], preferred_element_type=jnp.float32)
        + bm1_ref[...], 0.0)
    h = jnp.maximum(
        jnp.dot(h, wm2_ref[...], preferred_element_type=jnp.float32)
        + bm2_ref[...], 0.0)
    score_ref[...] = (
        jnp.dot(h, wm3_ref[...], preferred_element_type=jnp.float32)
        + bm3_ref[...])

    es1_ref[...] = jnp.concatenate(
        [o1_ref[...][:, None, :], p1_ref[...]], axis=1)
    es2_ref[...] = jnp.concatenate(
        [o2_ref[...][:, None, :], p2_ref[...]], axis=1)
    eout_ref[...] = emb_ref[...]

    def wr1(b, carry):
        d = d1_ref[0, b]
        eout_ref[pl.ds(d, 1)] = es1_ref[pl.ds(b, 1)]
        return carry

    lax.fori_loop(0, B, wr1, 0)

    def wr2(b, carry):
        d = d2_ref[0, b]
        eout_ref[pl.ds(d, 1)] = es2_ref[pl.ds(b, 1)]
        return carry

    lax.fori_loop(0, B, wr2, 0)


def _row_blocked(n_arrays, nrows):
    grid = (nrows // _RB,)
    row_spec = lambda w: pl.BlockSpec((_RB, w), lambda i: (i, 0))
    full = lambda a, b: pl.BlockSpec((a, b), lambda i: (0, 0))
    return grid, row_spec, full


def _tc_pre(x_all, deg_all, wn, bn, wg1):
    grid, row, full = _row_blocked(2, NC * NP)
    return pl.pallas_call(
        _tc_pre_body,
        grid=grid,
        in_specs=[row(D), row(D), full(D, D), full(1, D), full(D, D)],
        out_specs=[row(D), row(8)],
        out_shape=[
            jax.ShapeDtypeStruct((NC * NP, D), jnp.float32),
            jax.ShapeDtypeStruct((NC * NP, 8), jnp.float32),
        ],
    )(x_all, deg_all, wn, bn, wg1)


def _tc_mid(acc_all, u_all, dinv_all, bg1, wg2):
    grid, row, full = _row_blocked(2, NC * NP)
    return pl.pallas_call(
        _tc_mid_body,
        grid=grid,
        in_specs=[row(D), row(D), row(8), full(1, D), full(D, D)],
        out_specs=row(D),
        out_shape=jax.ShapeDtypeStruct((NC * NP, D), jnp.float32),
    )(acc_all, u_all, dinv_all, bg1, wg2)


def _tc_pool(acc2, u2, dinv, bg2, batch2d, wpat):
    fix = lambda a, b: pl.BlockSpec((a, b), lambda i: (0, 0))
    return pl.pallas_call(
        _tc_pool_body,
        grid=(N // _NB,),
        in_specs=[
            pl.BlockSpec((_NB, D), lambda i: (i, 0)),
            pl.BlockSpec((_NB, D), lambda i: (i, 0)),
            pl.BlockSpec((_NB, 8), lambda i: (i, 0)),
            fix(1, D),
            pl.BlockSpec((1, 1, _NB), lambda i: (i, 0, 0)),
            fix(D, P),
        ],
        out_specs=[
            fix(B, D),
            pl.BlockSpec((B, P, D), lambda i: (0, 0, 0)),
        ],
        out_shape=[
            jax.ShapeDtypeStruct((B, D), jnp.float32),
            jax.ShapeDtypeStruct((B, P, D), jnp.float32),
        ],
    )(acc2, u2, dinv, bg2, batch2d, wpat)


def _tc_head(o1, o2, p1, p2, ddi2d, d1, d2, wm0a, wm0b, wm0c, bm0,
             wm1, bm1, wm2, bm2, wm3, bm3, emb):
    smem_spec = pl.BlockSpec(memory_space=pltpu.SMEM)
    any_spec = pl.BlockSpec()
    return pl.pallas_call(
        _tc_head_body,
        in_specs=[any_spec] * 5 + [smem_spec, smem_spec] + [any_spec] * 11,
        out_specs=[any_spec, any_spec],
        out_shape=[
            jax.ShapeDtypeStruct((B, 1), jnp.float32),
            jax.ShapeDtypeStruct((NDRUGS, P + 1, D), jnp.float32),
        ],
        scratch_shapes=[
            pltpu.VMEM((B, P + 1, D), jnp.float32),
            pltpu.VMEM((B, P + 1, D), jnp.float32),
        ],
    )(o1, o2, p1, p2, ddi2d, d1, d2, wm0a, wm0b, wm0c, bm0,
      wm1, bm1, wm2, bm2, wm3, bm3, emb)


def kernel(x1, edge_index1, batch1, drug1, x2, edge_index2, batch2, drug2,
           ddi_type, W_node, b_node, W_g1, b_g1, W_g2, b_g2, W_pat,
           W_m0, b_m0, W_m1, b_m1, W_m2, b_m2, W_m3, b_m3, emb):
    i32 = jnp.int32
    src_all = jnp.concatenate(
        [edge_index1[0].astype(i32), edge_index2[0].astype(i32) + NP])
    dst_all = jnp.concatenate(
        [edge_index1[1].astype(i32), edge_index2[1].astype(i32)])
    xpad = jnp.zeros((NP - N, D), jnp.float32)
    x_all = jnp.concatenate([x1, xpad, x2, xpad], axis=0)

    onesD = jnp.ones((K, D), jnp.float32)
    zzD = jnp.zeros((RPT, D), jnp.float32)

    deg_all = _sc_degree(dst_all, onesD, zzD)

    bn = b_node.reshape(1, D)
    bg1 = b_g1.reshape(1, D)
    bg2 = b_g2.reshape(1, D)
    u_all, dinv_all = _tc_pre(x_all, deg_all, W_node, bn, W_g1)

    acc1 = _sc_conv(u_all, src_all, dst_all, zzD)
    u2_all = _tc_mid(acc1, u_all, dinv_all, bg1, W_g2)
    acc2 = _sc_conv(u2_all, src_all, dst_all, zzD)

    b1_2d = batch1.astype(i32).reshape(N // _NB, 1, _NB)
    b2_2d = batch2.astype(i32).reshape(N // _NB, 1, _NB)
    out1, pool1 = _tc_pool(acc2[:N], u2_all[:N], dinv_all[:N], bg2,
                           b1_2d, W_pat)
    out2, pool2 = _tc_pool(acc2[NP:NP + N], u2_all[NP:NP + N],
                           dinv_all[NP:NP + N], bg2, b2_2d, W_pat)

    score2d, emb_out = _tc_head(
        out1, out2, pool1, pool2,
        ddi_type.astype(i32).reshape(B, 1),
        drug1.astype(i32).reshape(1, B),
        drug2.astype(i32).reshape(1, B),
        W_m0[:D], W_m0[D:2 * D], W_m0[2 * D:], b_m0.reshape(1, D),
        W_m1, b_m1.reshape(1, D), W_m2, b_m2.reshape(1, D),
        W_m3, b_m3.reshape(1, 1), emb)

    return score2d[:, 0], emb_out


# speedup vs baseline: 17.1516x; 1.2227x over previous
import functools

import jax
import jax.numpy as jnp
from jax import lax
from jax.experimental import pallas as pl
from jax.experimental.pallas import tpu as pltpu
import jax.experimental.pallas.tpu_sc as plsc

N = 10000
NP = 10240
E = 320000
D = 128
B = 256
P = 8
NDDI = 86
NDRUGS = 1000

NC = 2
NS = 16
EPT = E // NS
K = 80
NIT = EPT // K
RPT = NP // NS

_MESH = plsc.VectorSubcoreMesh(
    core_axis_name="c", subcore_axis_name="s", num_cores=NC, num_subcores=NS)


@functools.partial(
    pl.kernel,
    out_type=jax.ShapeDtypeStruct((NC * NP, D), jnp.float32),
    mesh=_MESH,
    scratch_types=[
        pltpu.VMEM((K,), jnp.int32),
        pltpu.VMEM((K, D), jnp.float32),
        pltpu.VMEM_SHARED((NP, D), jnp.float32),
    ],
)
def _sc_degree(dst_hbm, ones_hbm, zz_hbm, out_hbm, dst_v, ones_v, deg_sh):
    c = lax.axis_index("c")
    s = lax.axis_index("s")
    pltpu.sync_copy(ones_hbm, ones_v)
    pltpu.sync_copy(zz_hbm, deg_sh.at[pl.ds(s * RPT, RPT)])
    plsc.subcore_barrier()

    def body(i, carry):
        base = c * E + s * EPT + i * K
        pltpu.sync_copy(dst_hbm.at[pl.ds(base, K)], dst_v)
        pltpu.sync_copy(ones_v, deg_sh.at[dst_v], add=True)
        return carry

    lax.fori_loop(0, NIT, body, 0)
    plsc.subcore_barrier()
    pltpu.sync_copy(deg_sh.at[pl.ds(s * RPT, RPT)],
                    out_hbm.at[pl.ds(c * NP + s * RPT, RPT)])


CB = 10
NOUT = NIT // CB


@functools.partial(
    pl.kernel,
    out_type=jax.ShapeDtypeStruct((NC * NP, D), jnp.float32),
    mesh=_MESH,
    scratch_types=[
        pltpu.VMEM((CB, K), jnp.int32),
        pltpu.VMEM((CB, K), jnp.int32),
        pltpu.VMEM((K, D), jnp.float32),
        pltpu.VMEM((K, D), jnp.float32),
        pltpu.VMEM_SHARED((NP, D), jnp.float32),
        pltpu.SemaphoreType.DMA,
        pltpu.SemaphoreType.DMA,
        pltpu.SemaphoreType.DMA,
    ],
)
def _sc_conv(u_hbm, src_hbm, dst_hbm, zz_hbm, out_hbm,
             sidx, didx, rowsA, rowsB, acc_sh, semA, semB, semI):
    c = lax.axis_index("c")
    s = lax.axis_index("s")
    pltpu.sync_copy(zz_hbm, acc_sh.at[pl.ds(s * RPT, RPT)])
    plsc.subcore_barrier()

    rows = (rowsA, rowsB)
    sems = (semA, semB)
    base0 = c * E + s * EPT

    def outer(j, carry):
        bj = base0 + j * (CB * K)
        icps = []
        for r in range(CB):
            icps.append(pltpu.async_copy(
                src_hbm.at[pl.ds(bj + r * K, K)], sidx.at[r], semI))
            icps.append(pltpu.async_copy(
                dst_hbm.at[pl.ds(bj + r * K, K)], didx.at[r], semI))
        for cp in icps:
            cp.wait()
        cp = pltpu.async_copy(u_hbm.at[sidx.at[0]], rows[0], sems[0])
        for r in range(CB):
            if r + 1 < CB:
                cp_n = pltpu.async_copy(
                    u_hbm.at[sidx.at[r + 1]], rows[(r + 1) % 2],
                    sems[(r + 1) % 2])
            cp.wait()
            pltpu.sync_copy(rows[r % 2], acc_sh.at[didx.at[r]], add=True)
            if r + 1 < CB:
                cp = cp_n
        return carry

    lax.fori_loop(0, NOUT, outer, 0)

    plsc.subcore_barrier()
    pltpu.sync_copy(acc_sh.at[pl.ds(s * RPT, RPT)],
                    out_hbm.at[pl.ds(c * NP + s * RPT, RPT)])


_RB = 2048
_NB = 2000


def _tc_pre_body(x_ref, deg_ref, wn_ref, bn_ref, wg1_ref, u_ref, dinv_ref):
    h = jnp.dot(x_ref[...], wn_ref[...], preferred_element_type=jnp.float32)
    h = h + bn_ref[...]
    z = jnp.dot(h, wg1_ref[...], preferred_element_type=jnp.float32)
    dinv = lax.rsqrt(deg_ref[...][:, 0:8] + 1.0)
    dinv_ref[...] = dinv
    u_ref[...] = z * dinv[:, 0:1]


def _tc_mid_body(acc_ref, u_ref, dinv_ref, bg_ref, wg2_ref, u2_ref):
    dv = dinv_ref[...][:, 0:1]
    a = dv * (acc_ref[...] + u_ref[...]) + bg_ref[...]
    z2 = jnp.dot(a, wg2_ref[...], preferred_element_type=jnp.float32)
    u2_ref[...] = z2 * dv


def _tc_pool_body(acc_ref, u_ref, dinv_ref, bg_ref, batch_ref, wpat_ref,
                  out_ref, pool_ref):
    i = pl.program_id(0)
    hf = dinv_ref[...][:, 0:1] * (acc_ref[...] + u_ref[...]) + bg_ref[...]
    seg = (lax.broadcasted_iota(jnp.int32, (B, _NB), 0)
           == batch_ref[0]).astype(jnp.float32)
    part_out = jnp.dot(seg, hf, preferred_element_type=jnp.float32,
                       precision=lax.Precision.HIGHEST)
    sc = jnp.dot(hf, wpat_ref[...], preferred_element_type=jnp.float32)
    sc = sc - jnp.max(sc, axis=1, keepdims=True)
    ex = jnp.exp(sc)
    attn = ex / jnp.sum(ex, axis=1, keepdims=True)
    pools = [
        jnp.dot(seg, attn[:, p:p + 1] * hf, preferred_element_type=jnp.float32,
                precision=lax.Precision.HIGHEST)
        for p in range(P)
    ]
    pstk = jnp.stack(pools, axis=1)

    @pl.when(i == 0)
    def _():
        out_ref[...] = part_out
        pool_ref[...] = pstk

    @pl.when(i > 0)
    def _():
        out_ref[...] = out_ref[...] + part_out
        pool_ref[...] = pool_ref[...] + pstk

    @pl.when(i == (N // _NB) - 1)
    def _():
        pv = pool_ref[...]
        nrm = jnp.sqrt(jnp.sum(pv * pv, axis=-1, keepdims=True))
        pool_ref[...] = pv / jnp.maximum(nrm, 1e-12)


def _tc_head_body(o1_ref, o2_ref, p1_ref, p2_ref, ddi_ref, d1_ref, d2_ref,
                  wm0a_ref, wm0b_ref, wm0c_ref, bm0_ref, wm1_ref, bm1_ref,
                  wm2_ref, bm2_ref, wm3_ref, bm3_ref, emb_ref,
                  score_ref, eout_ref, es1_ref, es2_ref):
    oh = (lax.broadcasted_iota(jnp.int32, (B, NDDI), 1)
          == ddi_ref[...]).astype(jnp.float32)
    h = (jnp.dot(o1_ref[...], wm0a_ref[...], preferred_element_type=jnp.float32)
         + jnp.dot(o2_ref[...], wm0b_ref[...], preferred_element_type=jnp.float32)
         + jnp.dot(oh, wm0c_ref[...], preferred_element_type=jnp.float32)
         + bm0_ref[...])
    h = jnp.maximum(
        jnp.dot(h, wm1_ref[...], preferred_element_type=jnp.float32)
        + bm1_ref[...], 0.0)
    h = jnp.maximum(
        jnp.dot(h, wm2_ref[...], preferred_element_type=jnp.float32)
        + bm2_ref[...], 0.0)
    score_ref[...] = (
        jnp.dot(h, wm3_ref[...], preferred_element_type=jnp.float32)
        + bm3_ref[...])

    es1_ref[...] = jnp.concatenate(
        [o1_ref[...][:, None, :], p1_ref[...]], axis=1)
    es2_ref[...] = jnp.concatenate(
        [o2_ref[...][:, None, :], p2_ref[...]], axis=1)
    eout_ref[...] = emb_ref[...]

    def wr1(b, carry):
        d = d1_ref[0, b]
        eout_ref[pl.ds(d, 1)] = es1_ref[pl.ds(b, 1)]
        return carry

    lax.fori_loop(0, B, wr1, 0)

    def wr2(b, carry):
        d = d2_ref[0, b]
        eout_ref[pl.ds(d, 1)] = es2_ref[pl.ds(b, 1)]
        return carry

    lax.fori_loop(0, B, wr2, 0)


def _row_blocked(n_arrays, nrows):
    grid = (nrows // _RB,)
    row_spec = lambda w: pl.BlockSpec((_RB, w), lambda i: (i, 0))
    full = lambda a, b: pl.BlockSpec((a, b), lambda i: (0, 0))
    return grid, row_spec, full


def _tc_pre(x_all, deg_all, wn, bn, wg1):
    grid, row, full = _row_blocked(2, NC * NP)
    return pl.pallas_call(
        _tc_pre_body,
        grid=grid,
        in_specs=[row(D), row(D), full(D, D), full(1, D), full(D, D)],
        out_specs=[row(D), row(8)],
        out_shape=[
            jax.ShapeDtypeStruct((NC * NP, D), jnp.float32),
            jax.ShapeDtypeStruct((NC * NP, 8), jnp.float32),
        ],
    )(x_all, deg_all, wn, bn, wg1)


def _tc_mid(acc_all, u_all, dinv_all, bg1, wg2):
    grid, row, full = _row_blocked(2, NC * NP)
    return pl.pallas_call(
        _tc_mid_body,
        grid=grid,
        in_specs=[row(D), row(D), row(8), full(1, D), full(D, D)],
        out_specs=row(D),
        out_shape=jax.ShapeDtypeStruct((NC * NP, D), jnp.float32),
    )(acc_all, u_all, dinv_all, bg1, wg2)


def _tc_pool(acc2, u2, dinv, bg2, batch2d, wpat):
    fix = lambda a, b: pl.BlockSpec((a, b), lambda i: (0, 0))
    return pl.pallas_call(
        _tc_pool_body,
        grid=(N // _NB,),
        in_specs=[
            pl.BlockSpec((_NB, D), lambda i: (i, 0)),
            pl.BlockSpec((_NB, D), lambda i: (i, 0)),
            pl.BlockSpec((_NB, 8), lambda i: (i, 0)),
            fix(1, D),
            pl.BlockSpec((1, 1, _NB), lambda i: (i, 0, 0)),
            fix(D, P),
        ],
        out_specs=[
            fix(B, D),
            pl.BlockSpec((B, P, D), lambda i: (0, 0, 0)),
        ],
        out_shape=[
            jax.ShapeDtypeStruct((B, D), jnp.float32),
            jax.ShapeDtypeStruct((B, P, D), jnp.float32),
        ],
    )(acc2, u2, dinv, bg2, batch2d, wpat)


def _tc_head(o1, o2, p1, p2, ddi2d, d1, d2, wm0a, wm0b, wm0c, bm0,
             wm1, bm1, wm2, bm2, wm3, bm3, emb):
    smem_spec = pl.BlockSpec(memory_space=pltpu.SMEM)
    any_spec = pl.BlockSpec()
    return pl.pallas_call(
        _tc_head_body,
        in_specs=[any_spec] * 5 + [smem_spec, smem_spec] + [any_spec] * 11,
        out_specs=[any_spec, any_spec],
        out_shape=[
            jax.ShapeDtypeStruct((B, 1), jnp.float32),
            jax.ShapeDtypeStruct((NDRUGS, P + 1, D), jnp.float32),
        ],
        scratch_shapes=[
            pltpu.VMEM((B, P + 1, D), jnp.float32),
            pltpu.VMEM((B, P + 1, D), jnp.float32),
        ],
    )(o1, o2, p1, p2, ddi2d, d1, d2, wm0a, wm0b, wm0c, bm0,
      wm1, bm1, wm2, bm2, wm3, bm3, emb)


def kernel(x1, edge_index1, batch1, drug1, x2, edge_index2, batch2, drug2,
           ddi_type, W_node, b_node, W_g1, b_g1, W_g2, b_g2, W_pat,
           W_m0, b_m0, W_m1, b_m1, W_m2, b_m2, W_m3, b_m3, emb):
    i32 = jnp.int32
    src_all = jnp.concatenate(
        [edge_index1[0].astype(i32), edge_index2[0].astype(i32) + NP])
    dst_all = jnp.concatenate(
        [edge_index1[1].astype(i32), edge_index2[1].astype(i32)])
    xpad = jnp.zeros((NP - N, D), jnp.float32)
    x_all = jnp.concatenate([x1, xpad, x2, xpad], axis=0)

    onesD = jnp.ones((K, D), jnp.float32)
    zzD = jnp.zeros((RPT, D), jnp.float32)

    deg_all = _sc_degree(dst_all, onesD, zzD)

    bn = b_node.reshape(1, D)
    bg1 = b_g1.reshape(1, D)
    bg2 = b_g2.reshape(1, D)
    u_all, dinv_all = _tc_pre(x_all, deg_all, W_node, bn, W_g1)

    acc1 = _sc_conv(u_all, src_all, dst_all, zzD)
    u2_all = _tc_mid(acc1, u_all, dinv_all, bg1, W_g2)
    acc2 = _sc_conv(u2_all, src_all, dst_all, zzD)

    b1_2d = batch1.astype(i32).reshape(N // _NB, 1, _NB)
    b2_2d = batch2.astype(i32).reshape(N // _NB, 1, _NB)
    out1, pool1 = _tc_pool(acc2[:N], u2_all[:N], dinv_all[:N], bg2,
                           b1_2d, W_pat)
    out2, pool2 = _tc_pool(acc2[NP:NP + N], u2_all[NP:NP + N],
                           dinv_all[NP:NP + N], bg2, b2_2d, W_pat)

    score2d, emb_out = _tc_head(
        out1, out2, pool1, pool2,
        ddi_type.astype(i32).reshape(B, 1),
        drug1.astype(i32).reshape(1, B),
        drug2.astype(i32).reshape(1, B),
        W_m0[:D], W_m0[D:2 * D], W_m0[2 * D:], b_m0.reshape(1, D),
        W_m1, b_m1.reshape(1, D), W_m2, b_m2.reshape(1, D),
        W_m3, b_m3.reshape(1, 1), emb)

    return score2d[:, 0], emb_out
